# baseline (speedup 1.0000x reference)
import functools

import jax
import jax.numpy as jnp
from jax import lax
from jax.experimental import pallas as pl
from jax.experimental.pallas import tpu as pltpu
from jax.experimental.pallas import tpu_sc as plsc

NT = 32
NC = 2
NS = 16
CH = 128

G = 64


def _mesh():
    return plsc.VectorSubcoreMesh(
        core_axis_name="c", subcore_axis_name="s",
        num_cores=NC, num_subcores=NS)


def _make_deg_kernel(NP, NCH, RPT, H):
    @functools.partial(
        pl.kernel,
        out_type=jax.ShapeDtypeStruct((NC, NP, H), jnp.float32),
        mesh=_mesh(),
        scratch_types=[
            pltpu.VMEM_SHARED((NP, H), jnp.float32),
            pltpu.VMEM((NCH, CH), jnp.int32),
            pltpu.VMEM((CH, H), jnp.float32),
        ],
    )
    def deg_kernel(dstp_hbm, zh_hbm, ones_hbm, out_hbm, acc, dstbuf, ones_v):
        cid = lax.axis_index("c")
        sid = lax.axis_index("s")
        wid = sid * NC + cid
        r0 = sid * RPT
        pltpu.sync_copy(zh_hbm.at[pl.ds(r0, RPT)], acc.at[pl.ds(r0, RPT)])
        pltpu.sync_copy(ones_hbm, ones_v)
        pltpu.sync_copy(dstp_hbm.at[wid], dstbuf)
        plsc.subcore_barrier()

        def body(j, carry):
            pltpu.sync_copy(ones_v, acc.at[dstbuf.at[j]], add=True)
            return carry

        lax.fori_loop(0, NCH, body, 0)
        plsc.subcore_barrier()
        pltpu.sync_copy(acc.at[pl.ds(r0, RPT)],
                        out_hbm.at[cid].at[pl.ds(r0, RPT)])

    return deg_kernel


def _make_agg_kernel(NP, NCH, RPT, H):
    HNCH = NCH // 2

    @functools.partial(
        pl.kernel,
        out_type=jax.ShapeDtypeStruct((NC, NP, H), jnp.float32),
        mesh=_mesh(),
        scratch_types=[
            pltpu.VMEM_SHARED((NP, H), jnp.float32),
            pltpu.VMEM((HNCH, CH), jnp.int32),
            pltpu.VMEM((HNCH, CH), jnp.int32),
            pltpu.VMEM((CH, H), jnp.float32),
            pltpu.VMEM((CH, H), jnp.float32),
            pltpu.SemaphoreType.DMA,
            pltpu.SemaphoreType.DMA,
        ],
    )
    def agg_kernel(hp_hbm, srcp_hbm, dstp_hbm, zh_hbm, out_hbm,
                   acc, srcbuf, dstbuf, rows0, rows1, sem0, sem1):
        cid = lax.axis_index("c")
        sid = lax.axis_index("s")
        wid = sid * NC + cid
        r0 = sid * RPT
        pltpu.sync_copy(zh_hbm.at[pl.ds(r0, RPT)], acc.at[pl.ds(r0, RPT)])
        pltpu.sync_copy(srcp_hbm.at[wid].at[pl.ds(0, HNCH)], srcbuf)
        pltpu.sync_copy(dstp_hbm.at[wid].at[pl.ds(0, HNCH)], dstbuf)
        plsc.subcore_barrier()

        def body(t, c):
            @pl.when(2 * t == HNCH)
            def _():
                pltpu.sync_copy(srcp_hbm.at[wid].at[pl.ds(HNCH, HNCH)], srcbuf)
                pltpu.sync_copy(dstp_hbm.at[wid].at[pl.ds(HNCH, HNCH)], dstbuf)

            j0 = lax.rem(2 * t, HNCH)
            j1 = j0 + 1
            g0 = pltpu.async_copy(hp_hbm.at[srcbuf.at[j0]], rows0, sem0)
            g1 = pltpu.async_copy(hp_hbm.at[srcbuf.at[j1]], rows1, sem1)
            g0.wait()
            pltpu.sync_copy(rows0, acc.at[dstbuf.at[j0]], add=True)
            g1.wait()
            pltpu.sync_copy(rows1, acc.at[dstbuf.at[j1]], add=True)
            return c

        lax.fori_loop(0, NCH // 2, body, 0)
        plsc.subcore_barrier()
        pltpu.sync_copy(acc.at[pl.ds(r0, RPT)], out_hbm.at[cid].at[pl.ds(r0, RPT)])

    return agg_kernel


def _dis_from(deg2_ref):
    deg = deg2_ref[0, :, 0:1] + deg2_ref[1, :, 0:1] + 1.0
    return lax.rsqrt(deg)


def _tc_pre_body(N, NP, deg2_ref, x_ref, w_ref, hp_ref):
    dis = _dis_from(deg2_ref)
    h = jnp.dot(x_ref[...], w_ref[...], preferred_element_type=jnp.float32)
    hp_ref[pl.ds(0, N), :] = dis[:N] * h
    hp_ref[pl.ds(N, NP - N), :] = jnp.zeros((NP - N, h.shape[1]), jnp.float32)


def _tc_mid_body(N, NP, acc2_ref, hp_ref, deg2_ref, b_ref, w_ref, out_ref):
    dis = _dis_from(deg2_ref)
    agg = acc2_ref[0] + acc2_ref[1] + hp_ref[...]
    xn = jnp.maximum(agg * dis + b_ref[...], 0.0)
    h = jnp.dot(xn, w_ref[...], preferred_element_type=jnp.float32)
    out_ref[...] = dis * h
    out_ref[pl.ds(N, NP - N), :] = jnp.zeros((NP - N, h.shape[1]), jnp.float32)


def _tc_fin_body(N, acc2_ref, hp_ref, deg2_ref, b_ref, batch_ref, wl_ref,
                 bl_ref, out_ref):
    dis = _dis_from(deg2_ref)
    agg = acc2_ref[0] + acc2_ref[1] + hp_ref[...]
    x3 = jnp.maximum(agg * dis + b_ref[...], 0.0)[:N]
    gid = lax.broadcasted_iota(jnp.int32, (G, N), 0)
    oh = (gid == batch_ref[...]).astype(jnp.float32)
    sums = jnp.dot(oh, x3, preferred_element_type=jnp.float32)
    cnt = jnp.sum(oh, axis=1, keepdims=True)
    pooled = sums / jnp.maximum(cnt, 1.0)
    out_ref[...] = (
        jnp.dot(pooled, wl_ref[...], preferred_element_type=jnp.float32)
        + bl_ref[...]
    )


def kernel(x, edge_index, batch, W1, b1, W2, b2, W3, b3, Wl, bl):
    N, D = x.shape
    H = W1.shape[1]
    C = Wl.shape[1]
    E = edge_index.shape[1]

    NP = -(-(N + 1) // 128) * 128
    RPT = NP // NS
    NCH = -(-E // (NT * CH))
    NCH = -(-NCH // 4) * 4
    EP = NT * NCH * CH

    src = edge_index[0]
    dst = edge_index[1]
    padv = jnp.full((EP - E,), N, dtype=jnp.int32)
    srcp = jnp.concatenate([src, padv]).reshape(NT, NCH, CH)
    dstp = jnp.concatenate([dst, padv]).reshape(NT, NCH, CH)
    zh = jnp.zeros((NP, H), jnp.float32)
    onesh = jnp.ones((CH, H), jnp.float32)

    deg_k = _make_deg_kernel(NP, NCH, RPT, H)
    agg_k = _make_agg_kernel(NP, NCH, RPT, H)

    tc_pre = pl.pallas_call(
        functools.partial(_tc_pre_body, N, NP),
        out_shape=jax.ShapeDtypeStruct((NP, H), jnp.float32),
    )
    tc_mid = pl.pallas_call(
        functools.partial(_tc_mid_body, N, NP),
        out_shape=jax.ShapeDtypeStruct((NP, H), jnp.float32),
    )
    tc_fin = pl.pallas_call(
        functools.partial(_tc_fin_body, N),
        out_shape=jax.ShapeDtypeStruct((G, C), jnp.float32),
    )

    batch2 = batch.reshape(1, N)
    b1r, b2r, b3r, blr = (v.reshape(1, -1) for v in (b1, b2, b3, bl))

    deg2 = deg_k(dstp, zh, onesh)[:, :, :8]
    hp1 = tc_pre(deg2, x, W1)
    acc1 = agg_k(hp1, srcp, dstp, zh)
    hp2 = tc_mid(acc1, hp1, deg2, b1r, W2)
    acc2 = agg_k(hp2, srcp, dstp, zh)
    hp3 = tc_mid(acc2, hp2, deg2, b2r, W3)
    acc3 = agg_k(hp3, srcp, dstp, zh)
    return tc_fin(acc3, hp3, deg2, b3r, batch2, Wl, blr)


# speedup vs baseline: 1.0666x; 1.0666x over previous
import functools

import jax
import jax.numpy as jnp
from jax import lax
from jax.experimental import pallas as pl
from jax.experimental.pallas import tpu as pltpu
from jax.experimental.pallas import tpu_sc as plsc

NT = 32
NC = 2
NS = 16
CH = 256

G = 64


def _mesh():
    return plsc.VectorSubcoreMesh(
        core_axis_name="c", subcore_axis_name="s",
        num_cores=NC, num_subcores=NS)


def _make_deg_kernel(NP, NCH, RPT, H):
    @functools.partial(
        pl.kernel,
        out_type=jax.ShapeDtypeStruct((NC, NP, H), jnp.float32),
        mesh=_mesh(),
        scratch_types=[
            pltpu.VMEM_SHARED((NP, H), jnp.float32),
            pltpu.VMEM((NCH * CH,), jnp.int32),
            pltpu.VMEM((CH, H), jnp.float32),
        ],
    )
    def deg_kernel(dstp_hbm, zh_hbm, ones_hbm, out_hbm, acc, dstbuf, ones_v):
        cid = lax.axis_index("c")
        sid = lax.axis_index("s")
        wid = sid * NC + cid
        r0 = sid * RPT
        pltpu.sync_copy(zh_hbm.at[pl.ds(r0, RPT)], acc.at[pl.ds(r0, RPT)])
        pltpu.sync_copy(ones_hbm, ones_v)
        pltpu.sync_copy(dstp_hbm.at[wid], dstbuf)
        plsc.subcore_barrier()

        def body(j, carry):
            pltpu.sync_copy(ones_v, acc.at[dstbuf.at[pl.ds(j * CH, CH)]],
                            add=True)
            return carry

        lax.fori_loop(0, NCH, body, 0)
        plsc.subcore_barrier()
        pltpu.sync_copy(acc.at[pl.ds(r0, RPT)],
                        out_hbm.at[cid].at[pl.ds(r0, RPT)])

    return deg_kernel


def _make_agg_kernel(NP, NCH, RPT, H):
    HNCH = NCH // 2

    @functools.partial(
        pl.kernel,
        out_type=jax.ShapeDtypeStruct((NC, NP, H), jnp.float32),
        mesh=_mesh(),
        scratch_types=[
            pltpu.VMEM_SHARED((NP, H), jnp.float32),
            pltpu.VMEM((HNCH * CH,), jnp.int32),
            pltpu.VMEM((HNCH * CH,), jnp.int32),
            pltpu.VMEM((CH, H), jnp.float32),
            pltpu.SemaphoreType.DMA,
        ],
    )
    def agg_kernel(hp_hbm, srcp_hbm, dstp_hbm, zh_hbm, out_hbm,
                   acc, srcbuf, dstbuf, rows0, sem0):
        cid = lax.axis_index("c")
        sid = lax.axis_index("s")
        wid = sid * NC + cid
        r0 = sid * RPT
        pltpu.sync_copy(zh_hbm.at[pl.ds(r0, RPT)], acc.at[pl.ds(r0, RPT)])
        plsc.subcore_barrier()

        def body(j, c):
            pltpu.async_copy(
                hp_hbm.at[srcbuf.at[pl.ds(j * CH, CH)]], rows0, sem0).wait()
            pltpu.sync_copy(rows0, acc.at[dstbuf.at[pl.ds(j * CH, CH)]],
                            add=True)
            return c

        for ph in range(2):
            pltpu.sync_copy(
                srcp_hbm.at[wid].at[pl.ds(ph * HNCH * CH, HNCH * CH)], srcbuf)
            pltpu.sync_copy(
                dstp_hbm.at[wid].at[pl.ds(ph * HNCH * CH, HNCH * CH)], dstbuf)
            lax.fori_loop(0, HNCH, body, 0)
        plsc.subcore_barrier()
        pltpu.sync_copy(acc.at[pl.ds(r0, RPT)], out_hbm.at[cid].at[pl.ds(r0, RPT)])

    return agg_kernel


def _dis_from(deg2_ref):
    deg = deg2_ref[0, :, 0:1] + deg2_ref[1, :, 0:1] + 1.0
    return lax.rsqrt(deg)


def _tc_pre_body(N, NP, deg2_ref, x_ref, w_ref, hp_ref):
    dis = _dis_from(deg2_ref)
    h = jnp.dot(x_ref[...], w_ref[...], preferred_element_type=jnp.float32)
    hp_ref[pl.ds(0, N), :] = dis[:N] * h
    hp_ref[pl.ds(N, NP - N), :] = jnp.zeros((NP - N, h.shape[1]), jnp.float32)


def _tc_mid_body(N, NP, acc2_ref, hp_ref, deg2_ref, b_ref, w_ref, out_ref):
    dis = _dis_from(deg2_ref)
    agg = acc2_ref[0] + acc2_ref[1] + hp_ref[...]
    xn = jnp.maximum(agg * dis + b_ref[...], 0.0)
    h = jnp.dot(xn, w_ref[...], preferred_element_type=jnp.float32)
    out_ref[...] = dis * h
    out_ref[pl.ds(N, NP - N), :] = jnp.zeros((NP - N, h.shape[1]), jnp.float32)


def _tc_fin_body(N, acc2_ref, hp_ref, deg2_ref, b_ref, batch_ref, wl_ref,
                 bl_ref, out_ref):
    dis = _dis_from(deg2_ref)
    agg = acc2_ref[0] + acc2_ref[1] + hp_ref[...]
    x3 = jnp.maximum(agg * dis + b_ref[...], 0.0)[:N]
    gid = lax.broadcasted_iota(jnp.int32, (G, N), 0)
    oh = (gid == batch_ref[...]).astype(jnp.float32)
    sums = jnp.dot(oh, x3, preferred_element_type=jnp.float32)
    cnt = jnp.sum(oh, axis=1, keepdims=True)
    pooled = sums / jnp.maximum(cnt, 1.0)
    out_ref[...] = (
        jnp.dot(pooled, wl_ref[...], preferred_element_type=jnp.float32)
        + bl_ref[...]
    )


def kernel(x, edge_index, batch, W1, b1, W2, b2, W3, b3, Wl, bl):
    N, D = x.shape
    H = W1.shape[1]
    C = Wl.shape[1]
    E = edge_index.shape[1]

    NP = -(-(N + 1) // 128) * 128
    RPT = NP // NS
    NCH = -(-E // (NT * CH))
    NCH = -(-NCH // 2) * 2
    EP = NT * NCH * CH

    src = edge_index[0]
    dst = edge_index[1]
    padv = jnp.full((EP - E,), N, dtype=jnp.int32)
    srcp = jnp.concatenate([src, padv]).reshape(NT, NCH * CH)
    dstp = jnp.concatenate([dst, padv]).reshape(NT, NCH * CH)
    zh = jnp.zeros((NP, H), jnp.float32)
    onesh = jnp.ones((CH, H), jnp.float32)

    deg_k = _make_deg_kernel(NP, NCH, RPT, H)
    agg_k = _make_agg_kernel(NP, NCH, RPT, H)

    tc_pre = pl.pallas_call(
        functools.partial(_tc_pre_body, N, NP),
        out_shape=jax.ShapeDtypeStruct((NP, H), jnp.float32),
    )
    tc_mid = pl.pallas_call(
        functools.partial(_tc_mid_body, N, NP),
        out_shape=jax.ShapeDtypeStruct((NP, H), jnp.float32),
    )
    tc_fin = pl.pallas_call(
        functools.partial(_tc_fin_body, N),
        out_shape=jax.ShapeDtypeStruct((G, C), jnp.float32),
    )

    batch2 = batch.reshape(1, N)
    b1r, b2r, b3r, blr = (v.reshape(1, -1) for v in (b1, b2, b3, bl))

    deg2 = deg_k(dstp, zh, onesh)[:, :, :8]
    hp1 = tc_pre(deg2, x, W1)
    acc1 = agg_k(hp1, srcp, dstp, zh)
    hp2 = tc_mid(acc1, hp1, deg2, b1r, W2)
    acc2 = agg_k(hp2, srcp, dstp, zh)
    hp3 = tc_mid(acc2, hp2, deg2, b2r, W3)
    acc3 = agg_k(hp3, srcp, dstp, zh)
    return tc_fin(acc3, hp3, deg2, b3r, batch2, Wl, blr)


# speedup vs baseline: 1.4697x; 1.3780x over previous
import functools

import jax
import jax.numpy as jnp
from jax import lax
from jax.experimental import pallas as pl
from jax.experimental.pallas import tpu as pltpu
from jax.experimental.pallas import tpu_sc as plsc

NT = 32
NC = 2
NS = 16
CH = 128

G = 64


def _mesh():
    return plsc.VectorSubcoreMesh(
        core_axis_name="c", subcore_axis_name="s",
        num_cores=NC, num_subcores=NS)


def _make_deg_kernel(NP, NCH, RPT, H):
    @functools.partial(
        pl.kernel,
        out_type=jax.ShapeDtypeStruct((NC, NP, H), jnp.float32),
        mesh=_mesh(),
        scratch_types=[
            pltpu.VMEM_SHARED((NP, H), jnp.float32),
            pltpu.VMEM((NCH, CH), jnp.int32),
            pltpu.VMEM((CH, H), jnp.float32),
        ],
    )
    def deg_kernel(dstp_hbm, zh_hbm, ones_hbm, out_hbm, acc, dstbuf, ones_v):
        cid = lax.axis_index("c")
        sid = lax.axis_index("s")
        wid = sid * NC + cid
        r0 = sid * RPT
        pltpu.sync_copy(zh_hbm.at[pl.ds(r0, RPT)], acc.at[pl.ds(r0, RPT)])
        pltpu.sync_copy(ones_hbm, ones_v)
        pltpu.sync_copy(dstp_hbm.at[wid], dstbuf)
        plsc.subcore_barrier()

        def body(j, carry):
            pltpu.sync_copy(ones_v, acc.at[dstbuf.at[j]], add=True)
            return carry

        lax.fori_loop(0, NCH, body, 0)
        plsc.subcore_barrier()
        pltpu.sync_copy(acc.at[pl.ds(r0, RPT)],
                        out_hbm.at[cid].at[pl.ds(r0, RPT)])

    return deg_kernel


def _make_agg_kernel(NP, NCH, RPT, H):
    @functools.partial(
        pl.kernel,
        out_type=jax.ShapeDtypeStruct((NC, NP, H), jnp.float32),
        mesh=_mesh(),
        scratch_types=[
            pltpu.VMEM_SHARED((NP, H), jnp.float32),
            pltpu.VMEM((NCH, CH), jnp.int32),
            pltpu.VMEM((NCH, CH), jnp.int32),
            pltpu.VMEM((CH, H), jnp.float32),
            pltpu.SemaphoreType.DMA,
        ],
    )
    def agg_kernel(hp_hbm, srcp_hbm, dstp_hbm, zh_hbm, out_hbm,
                   acc, srcbuf, dstbuf, rows, sem):
        cid = lax.axis_index("c")
        sid = lax.axis_index("s")
        wid = sid * NC + cid
        r0 = sid * RPT
        pltpu.sync_copy(zh_hbm.at[pl.ds(r0, RPT)], acc.at[pl.ds(r0, RPT)])
        pltpu.sync_copy(srcp_hbm.at[wid], srcbuf)
        pltpu.sync_copy(dstp_hbm.at[wid], dstbuf)
        plsc.subcore_barrier()

        def body(j, carry):
            pltpu.async_copy(hp_hbm.at[srcbuf.at[j]], rows, sem).wait()
            pltpu.sync_copy(rows, acc.at[dstbuf.at[j]], add=True)
            return carry

        lax.fori_loop(0, NCH, body, 0)
        plsc.subcore_barrier()
        pltpu.sync_copy(acc.at[pl.ds(r0, RPT)], out_hbm.at[cid].at[pl.ds(r0, RPT)])

    return agg_kernel


def _dis_from(deg2_ref):
    deg = deg2_ref[0, :, 0:1] + deg2_ref[1, :, 0:1] + 1.0
    return lax.rsqrt(deg)


def _tc_pre_body(N, NP, deg2_ref, x_ref, w_ref, hp_ref):
    dis = _dis_from(deg2_ref)
    h = jnp.dot(x_ref[...], w_ref[...], preferred_element_type=jnp.float32)
    hp_ref[pl.ds(0, N), :] = dis[:N] * h
    hp_ref[pl.ds(N, NP - N), :] = jnp.zeros((NP - N, h.shape[1]), jnp.float32)


def _tc_mid_body(N, NP, acc2_ref, hp_ref, deg2_ref, b_ref, w_ref, out_ref):
    dis = _dis_from(deg2_ref)
    agg = acc2_ref[0] + acc2_ref[1] + hp_ref[...]
    xn = jnp.maximum(agg * dis + b_ref[...], 0.0)
    h = jnp.dot(xn, w_ref[...], preferred_element_type=jnp.float32)
    out_ref[...] = dis * h
    out_ref[pl.ds(N, NP - N), :] = jnp.zeros((NP - N, h.shape[1]), jnp.float32)


def _tc_fin_body(N, acc2_ref, hp_ref, deg2_ref, b_ref, batch_ref, wl_ref,
                 bl_ref, out_ref):
    dis = _dis_from(deg2_ref)
    agg = acc2_ref[0] + acc2_ref[1] + hp_ref[...]
    x3 = jnp.maximum(agg * dis + b_ref[...], 0.0)[:N]
    gid = lax.broadcasted_iota(jnp.int32, (G, N), 0)
    oh = (gid == batch_ref[...]).astype(jnp.float32)
    sums = jnp.dot(oh, x3, preferred_element_type=jnp.float32)
    cnt = jnp.sum(oh, axis=1, keepdims=True)
    pooled = sums / jnp.maximum(cnt, 1.0)
    out_ref[...] = (
        jnp.dot(pooled, wl_ref[...], preferred_element_type=jnp.float32)
        + bl_ref[...]
    )


def kernel(x, edge_index, batch, W1, b1, W2, b2, W3, b3, Wl, bl):
    N, D = x.shape
    H = W1.shape[1]
    C = Wl.shape[1]
    E = edge_index.shape[1]

    NP = -(-(N + 1) // 128) * 128
    RPT = NP // NS
    NCH = -(-E // (NT * CH))
    EP = NT * NCH * CH

    src = edge_index[0]
    dst = edge_index[1]
    padv = jnp.full((EP - E,), N, dtype=jnp.int32)
    srcp = jnp.concatenate([src, padv]).reshape(NT, NCH, CH)
    dstp = jnp.concatenate([dst, padv]).reshape(NT, NCH, CH)
    zh = jnp.zeros((NP, H), jnp.float32)
    onesh = jnp.ones((CH, H), jnp.float32)

    deg_k = _make_deg_kernel(NP, NCH, RPT, H)
    agg_k = _make_agg_kernel(NP, NCH, RPT, H)

    tc_pre = pl.pallas_call(
        functools.partial(_tc_pre_body, N, NP),
        out_shape=jax.ShapeDtypeStruct((NP, H), jnp.float32),
    )
    tc_mid = pl.pallas_call(
        functools.partial(_tc_mid_body, N, NP),
        out_shape=jax.ShapeDtypeStruct((NP, H), jnp.float32),
    )
    tc_fin = pl.pallas_call(
        functools.partial(_tc_fin_body, N),
        out_shape=jax.ShapeDtypeStruct((G, C), jnp.float32),
    )

    batch2 = batch.reshape(1, N)
    b1r, b2r, b3r, blr = (v.reshape(1, -1) for v in (b1, b2, b3, bl))

    deg2 = deg_k(dstp, zh, onesh)[:, :, :8]
    hp1 = tc_pre(deg2, x, W1)
    acc1 = agg_k(hp1, srcp, dstp, zh)
    hp2 = tc_mid(acc1, hp1, deg2, b1r, W2)
    acc2 = agg_k(hp2, srcp, dstp, zh)
    hp3 = tc_mid(acc2, hp2, deg2, b2r, W3)
    acc3 = agg_k(hp3, srcp, dstp, zh)
    return tc_fin(acc3, hp3, deg2, b3r, batch2, Wl, blr)


# speedup vs baseline: 1.4750x; 1.0035x over previous
import functools

import jax
import jax.numpy as jnp
from jax import lax
from jax.experimental import pallas as pl
from jax.experimental.pallas import tpu as pltpu
from jax.experimental.pallas import tpu_sc as plsc

NT = 32
NC = 2
NS = 16
CH = 128

G = 64


def _mesh():
    return plsc.VectorSubcoreMesh(
        core_axis_name="c", subcore_axis_name="s",
        num_cores=NC, num_subcores=NS)


def _make_deg_kernel(NP, NCH, RPT, H):
    @functools.partial(
        pl.kernel,
        out_type=jax.ShapeDtypeStruct((NC, NP, H), jnp.float32),
        mesh=_mesh(),
        scratch_types=[
            pltpu.VMEM_SHARED((NP, H), jnp.float32),
            pltpu.VMEM((NCH, CH), jnp.int32),
            pltpu.VMEM((CH, H), jnp.float32),
        ],
    )
    def deg_kernel(dstp_hbm, zh_hbm, ones_hbm, out_hbm, acc, dstbuf, ones_v):
        cid = lax.axis_index("c")
        sid = lax.axis_index("s")
        wid = sid * NC + cid
        r0 = sid * RPT
        pltpu.sync_copy(zh_hbm.at[pl.ds(r0, RPT)], acc.at[pl.ds(r0, RPT)])
        pltpu.sync_copy(ones_hbm, ones_v)
        pltpu.sync_copy(dstp_hbm.at[wid], dstbuf)
        plsc.subcore_barrier()

        def body(j, carry):
            pltpu.sync_copy(ones_v, acc.at[dstbuf.at[j]], add=True)
            return carry

        lax.fori_loop(0, NCH, body, 0)
        plsc.subcore_barrier()
        pltpu.sync_copy(acc.at[pl.ds(r0, RPT)],
                        out_hbm.at[cid].at[pl.ds(r0, RPT)])

    return deg_kernel


def _make_agg_kernel(NP, NCH, RPT, H):
    @functools.partial(
        pl.kernel,
        out_type=jax.ShapeDtypeStruct((NC, NP, H), jnp.float32),
        mesh=_mesh(),
        scratch_types=[
            pltpu.VMEM_SHARED((NP, H), jnp.float32),
            pltpu.VMEM((NCH, CH), jnp.int32),
            pltpu.VMEM((NCH, CH), jnp.int32),
            pltpu.VMEM((CH, H), jnp.float32),
            pltpu.SemaphoreType.DMA,
        ],
    )
    def agg_kernel(hp_hbm, srcp_hbm, dstp_hbm, zh_hbm, out_hbm,
                   acc, srcbuf, dstbuf, rows, sem):
        cid = lax.axis_index("c")
        sid = lax.axis_index("s")
        wid = sid * NC + cid
        r0 = sid * RPT
        pltpu.sync_copy(zh_hbm.at[pl.ds(r0, RPT)], acc.at[pl.ds(r0, RPT)])
        pltpu.sync_copy(srcp_hbm.at[wid], srcbuf)
        pltpu.sync_copy(dstp_hbm.at[wid], dstbuf)
        plsc.subcore_barrier()

        def body(j, carry):
            pltpu.async_copy(hp_hbm.at[srcbuf.at[j]], rows, sem).wait()
            pltpu.sync_copy(rows, acc.at[dstbuf.at[j]], add=True)
            return carry

        lax.fori_loop(0, NCH, body, 0)
        plsc.subcore_barrier()
        pltpu.sync_copy(acc.at[pl.ds(r0, RPT)], out_hbm.at[cid].at[pl.ds(r0, RPT)])

    return agg_kernel


def _dis_from(deg2_ref):
    deg = deg2_ref[0, :, 0:1] + deg2_ref[1, :, 0:1] + 1.0
    return lax.rsqrt(deg)


def _tc_mm_body(x_ref, w_ref, h_ref):
    h_ref[...] = jnp.dot(x_ref[...], w_ref[...],
                         preferred_element_type=jnp.float32)


def _tc_pre_body(N, NP, deg2_ref, h_ref, hp_ref):
    dis = _dis_from(deg2_ref)
    hp_ref[pl.ds(0, N), :] = dis[:N] * h_ref[...]
    hp_ref[pl.ds(N, NP - N), :] = jnp.zeros(
        (NP - N, h_ref.shape[1]), jnp.float32)


def _tc_mid_body(N, NP, acc2_ref, hp_ref, deg2_ref, b_ref, w_ref, out_ref):
    dis = _dis_from(deg2_ref)
    agg = acc2_ref[0] + acc2_ref[1] + hp_ref[...]
    xn = jnp.maximum(agg * dis + b_ref[...], 0.0)
    h = jnp.dot(xn, w_ref[...], preferred_element_type=jnp.float32)
    out_ref[...] = dis * h
    out_ref[pl.ds(N, NP - N), :] = jnp.zeros((NP - N, h.shape[1]), jnp.float32)


def _tc_fin_body(N, acc2_ref, hp_ref, deg2_ref, b_ref, batch_ref, wl_ref,
                 bl_ref, out_ref):
    dis = _dis_from(deg2_ref)
    agg = acc2_ref[0] + acc2_ref[1] + hp_ref[...]
    x3 = jnp.maximum(agg * dis + b_ref[...], 0.0)[:N]
    gid = lax.broadcasted_iota(jnp.int32, (G, N), 0)
    oh = (gid == batch_ref[...]).astype(jnp.float32)
    sums = jnp.dot(oh, x3, preferred_element_type=jnp.float32)
    cnt = jnp.sum(oh, axis=1, keepdims=True)
    pooled = sums / jnp.maximum(cnt, 1.0)
    out_ref[...] = (
        jnp.dot(pooled, wl_ref[...], preferred_element_type=jnp.float32)
        + bl_ref[...]
    )


def kernel(x, edge_index, batch, W1, b1, W2, b2, W3, b3, Wl, bl):
    N, D = x.shape
    H = W1.shape[1]
    C = Wl.shape[1]
    E = edge_index.shape[1]

    NP = -(-(N + 1) // 128) * 128
    RPT = NP // NS
    NCH = -(-E // (NT * CH))
    EP = NT * NCH * CH

    src = edge_index[0]
    dst = edge_index[1]
    padv = jnp.full((EP - E,), N, dtype=jnp.int32)
    srcp = jnp.concatenate([src, padv]).reshape(NT, NCH, CH)
    dstp = jnp.concatenate([dst, padv]).reshape(NT, NCH, CH)
    zh = jnp.zeros((NP, H), jnp.float32)
    onesh = jnp.ones((CH, H), jnp.float32)

    deg_k = _make_deg_kernel(NP, NCH, RPT, H)
    agg_k = _make_agg_kernel(NP, NCH, RPT, H)

    tc_mm = pl.pallas_call(
        _tc_mm_body,
        out_shape=jax.ShapeDtypeStruct((N, H), jnp.float32),
    )
    tc_pre = pl.pallas_call(
        functools.partial(_tc_pre_body, N, NP),
        out_shape=jax.ShapeDtypeStruct((NP, H), jnp.float32),
    )
    tc_mid = pl.pallas_call(
        functools.partial(_tc_mid_body, N, NP),
        out_shape=jax.ShapeDtypeStruct((NP, H), jnp.float32),
    )
    tc_fin = pl.pallas_call(
        functools.partial(_tc_fin_body, N),
        out_shape=jax.ShapeDtypeStruct((G, C), jnp.float32),
    )

    batch2 = batch.reshape(1, N)
    b1r, b2r, b3r, blr = (v.reshape(1, -1) for v in (b1, b2, b3, bl))

    h1 = tc_mm(x, W1)
    deg2 = deg_k(dstp, zh, onesh)[:, :, :8]
    hp1 = tc_pre(deg2, h1)
    acc1 = agg_k(hp1, srcp, dstp, zh)
    hp2 = tc_mid(acc1, hp1, deg2, b1r, W2)
    acc2 = agg_k(hp2, srcp, dstp, zh)
    hp3 = tc_mid(acc2, hp2, deg2, b2r, W3)
    acc3 = agg_k(hp3, srcp, dstp, zh)
    return tc_fin(acc3, hp3, deg2, b3r, batch2, Wl, blr)


# speedup vs baseline: 1.4754x; 1.0003x over previous
import functools

import jax
import jax.numpy as jnp
from jax import lax
from jax.experimental import pallas as pl
from jax.experimental.pallas import tpu as pltpu
from jax.experimental.pallas import tpu_sc as plsc

NT = 32
NC = 2
NS = 16
CH = 128

G = 64


def _mesh():
    return plsc.VectorSubcoreMesh(
        core_axis_name="c", subcore_axis_name="s",
        num_cores=NC, num_subcores=NS)


def _make_deg_kernel(NP, NCH, RPT, H):
    @functools.partial(
        pl.kernel,
        out_type=jax.ShapeDtypeStruct((NC, NP, H), jnp.float32),
        mesh=_mesh(),
        scratch_types=[
            pltpu.VMEM_SHARED((NP, H), jnp.float32),
            pltpu.VMEM((NCH, CH), jnp.int32),
            pltpu.VMEM((CH, H), jnp.float32),
        ],
    )
    def deg_kernel(dstp_hbm, zh_hbm, ones_hbm, out_hbm, acc, dstbuf, ones_v):
        cid = lax.axis_index("c")
        sid = lax.axis_index("s")
        wid = sid * NC + cid
        r0 = sid * RPT
        pltpu.sync_copy(zh_hbm.at[pl.ds(r0, RPT)], acc.at[pl.ds(r0, RPT)])
        pltpu.sync_copy(ones_hbm, ones_v)
        pltpu.sync_copy(dstp_hbm.at[wid], dstbuf)
        plsc.subcore_barrier()

        def body(j, carry):
            pltpu.sync_copy(ones_v, acc.at[dstbuf.at[j]], add=True)
            return carry

        lax.fori_loop(0, NCH, body, 0)
        plsc.subcore_barrier()
        pltpu.sync_copy(acc.at[pl.ds(r0, RPT)],
                        out_hbm.at[cid].at[pl.ds(r0, RPT)])

    return deg_kernel


def _make_agg_kernel(NP, NCH, RPT, H):
    @functools.partial(
        pl.kernel,
        out_type=jax.ShapeDtypeStruct((NC, NP, H), jnp.float32),
        mesh=_mesh(),
        scratch_types=[
            pltpu.VMEM_SHARED((NP, H), jnp.float32),
            pltpu.VMEM((NCH, CH), jnp.int32),
            pltpu.VMEM((NCH, CH), jnp.int32),
            pltpu.VMEM((CH, H), jnp.float32),
            pltpu.SemaphoreType.DMA,
        ],
    )
    def agg_kernel(hp_hbm, srcp_hbm, dstp_hbm, zh_hbm, out_hbm,
                   acc, srcbuf, dstbuf, rows, sem):
        cid = lax.axis_index("c")
        sid = lax.axis_index("s")
        wid = sid * NC + cid
        r0 = sid * RPT
        pltpu.sync_copy(zh_hbm.at[pl.ds(r0, RPT)], acc.at[pl.ds(r0, RPT)])
        pltpu.sync_copy(srcp_hbm.at[wid], srcbuf)
        pltpu.sync_copy(dstp_hbm.at[wid], dstbuf)
        plsc.subcore_barrier()

        def body(j, carry):
            pltpu.sync_copy(hp_hbm.at[srcbuf.at[j]], rows)
            pltpu.sync_copy(rows, acc.at[dstbuf.at[j]], add=True)
            return carry

        lax.fori_loop(0, NCH, body, 0)
        plsc.subcore_barrier()
        pltpu.sync_copy(acc.at[pl.ds(r0, RPT)], out_hbm.at[cid].at[pl.ds(r0, RPT)])

    return agg_kernel


def _dis_from(deg2_ref):
    deg = deg2_ref[0, :, 0:1] + deg2_ref[1, :, 0:1] + 1.0
    return lax.rsqrt(deg)


def _tc_mm_body(x_ref, w_ref, h_ref):
    h_ref[...] = jnp.dot(x_ref[...], w_ref[...],
                         preferred_element_type=jnp.float32)


def _tc_pre_body(N, NP, deg2_ref, h_ref, hp_ref):
    dis = _dis_from(deg2_ref)
    hp_ref[pl.ds(0, N), :] = dis[:N] * h_ref[...]
    hp_ref[pl.ds(N, NP - N), :] = jnp.zeros(
        (NP - N, h_ref.shape[1]), jnp.float32)


def _tc_mid_body(N, NP, acc2_ref, hp_ref, deg2_ref, b_ref, w_ref, out_ref):
    dis = _dis_from(deg2_ref)
    agg = acc2_ref[0] + acc2_ref[1] + hp_ref[...]
    xn = jnp.maximum(agg * dis + b_ref[...], 0.0)
    h = jnp.dot(xn, w_ref[...], preferred_element_type=jnp.float32)
    out_ref[...] = dis * h
    out_ref[pl.ds(N, NP - N), :] = jnp.zeros((NP - N, h.shape[1]), jnp.float32)


def _tc_fin_body(N, acc2_ref, hp_ref, deg2_ref, b_ref, batch_ref, wl_ref,
                 bl_ref, out_ref):
    dis = _dis_from(deg2_ref)
    agg = acc2_ref[0] + acc2_ref[1] + hp_ref[...]
    x3 = jnp.maximum(agg * dis + b_ref[...], 0.0)[:N]
    gid = lax.broadcasted_iota(jnp.int32, (G, N), 0)
    oh = (gid == batch_ref[...]).astype(jnp.float32)
    sums = jnp.dot(oh, x3, preferred_element_type=jnp.float32)
    cnt = jnp.sum(oh, axis=1, keepdims=True)
    pooled = sums / jnp.maximum(cnt, 1.0)
    out_ref[...] = (
        jnp.dot(pooled, wl_ref[...], preferred_element_type=jnp.float32)
        + bl_ref[...]
    )


def kernel(x, edge_index, batch, W1, b1, W2, b2, W3, b3, Wl, bl):
    N, D = x.shape
    H = W1.shape[1]
    C = Wl.shape[1]
    E = edge_index.shape[1]

    NP = -(-(N + 1) // 128) * 128
    RPT = NP // NS
    NCH = -(-E // (NT * CH))
    EP = NT * NCH * CH

    src = edge_index[0]
    dst = edge_index[1]
    padv = jnp.full((EP - E,), N, dtype=jnp.int32)
    srcp = jnp.concatenate([src, padv]).reshape(NT, NCH, CH)
    dstp = jnp.concatenate([dst, padv]).reshape(NT, NCH, CH)
    zh = jnp.zeros((NP, H), jnp.float32)
    onesh = jnp.ones((CH, H), jnp.float32)

    deg_k = _make_deg_kernel(NP, NCH, RPT, H)
    agg_k = _make_agg_kernel(NP, NCH, RPT, H)

    tc_mm = pl.pallas_call(
        _tc_mm_body,
        out_shape=jax.ShapeDtypeStruct((N, H), jnp.float32),
    )
    tc_pre = pl.pallas_call(
        functools.partial(_tc_pre_body, N, NP),
        out_shape=jax.ShapeDtypeStruct((NP, H), jnp.float32),
    )
    tc_mid = pl.pallas_call(
        functools.partial(_tc_mid_body, N, NP),
        out_shape=jax.ShapeDtypeStruct((NP, H), jnp.float32),
    )
    tc_fin = pl.pallas_call(
        functools.partial(_tc_fin_body, N),
        out_shape=jax.ShapeDtypeStruct((G, C), jnp.float32),
    )

    batch2 = batch.reshape(1, N)
    b1r, b2r, b3r, blr = (v.reshape(1, -1) for v in (b1, b2, b3, bl))

    h1 = tc_mm(x, W1)
    deg2 = deg_k(dstp, zh, onesh)[:, :, :8]
    hp1 = tc_pre(deg2, h1)
    acc1 = agg_k(hp1, srcp, dstp, zh)
    hp2 = tc_mid(acc1, hp1, deg2, b1r, W2)
    acc2 = agg_k(hp2, srcp, dstp, zh)
    hp3 = tc_mid(acc2, hp2, deg2, b2r, W3)
    acc3 = agg_k(hp3, srcp, dstp, zh)
    return tc_fin(acc3, hp3, deg2, b3r, batch2, Wl, blr)
